# baseline (speedup 1.0000x reference)
import jax
import jax.numpy as jnp
from jax import lax
from jax.experimental import pallas as pl
from jax.experimental.pallas import tpu as pltpu
from jax.experimental.pallas import tpu_sc as plsc

B = 16384
VOCAB = 100000
EMB = 32
IDX_DIM = 64
HID = 256
U_FIELDS = 3
I_FIELDS = 4
N_FIELDS = U_FIELDS + I_FIELDS

NC = 2
NS = 16
NW = NC * NS

BW = B // NW
CHUNK = 128


def _sc_gather_body(u_tab, i_tab, u_feat, i_feat,
                    o0, o1, o2, o3, o4, o5, o6,
                    idx_v, rows_v, sem):
    wid = lax.axis_index("s") * NC + lax.axis_index("c")
    base = wid * BW

    def phase(tab_hbm, feat_hbm, out_hbm, f):
        pltpu.sync_copy(feat_hbm.at[f, pl.ds(base, BW)], idx_v)
        copies = []
        for c in range(BW // CHUNK):
            copies.append(pltpu.async_copy(
                tab_hbm.at[f].at[idx_v.at[pl.ds(c * CHUNK, CHUNK)]],
                rows_v.at[pl.ds(c * CHUNK, CHUNK)],
                sem,
            ))
        for cp in copies:
            cp.wait()
        pltpu.sync_copy(rows_v, out_hbm.at[pl.ds(base, BW)])

    for f, out in enumerate((o0, o1, o2)):
        phase(u_tab, u_feat, out, f)
    for f, out in enumerate((o3, o4, o5, o6)):
        phase(i_tab, i_feat, out, f)


def _mlp_body(idxT_ref, u0, u1, u2, i0, i1, i2, i3,
              w1_ref, b1_ref, w2_ref, b2_ref, o_ref):
    hp = jax.lax.Precision.HIGHEST

    def mm(x, lo, hi):
        return jnp.dot(x, w1_ref[lo:hi, :],
                       preferred_element_type=jnp.float32, precision=hp)

    h = lax.dot_general(idxT_ref[...], w1_ref[0:IDX_DIM, :],
                        dimension_numbers=(((0,), (0,)), ((), ())),
                        preferred_element_type=jnp.float32, precision=hp)
    for k, ref in enumerate((u0, u1, u2, i0, i1, i2, i3)):
        h += mm(ref[...], IDX_DIM + k * EMB, IDX_DIM + (k + 1) * EMB)
    h = jnp.maximum(h + b1_ref[...], 0.0)
    logits = jnp.dot(h, w2_ref[...],
                     preferred_element_type=jnp.float32, precision=hp)
    logits += b2_ref[...]
    m = jnp.max(logits, axis=-1, keepdims=True)
    e = jnp.exp(logits - m)
    o_ref[...] = e / jnp.sum(e, axis=-1, keepdims=True)


_MLP_BLK = 2048


def kernel(indexEmb, userFeatures, itemFeatures, user_table, item_table,
           W1, b1, W2, b2):
    u_feat = jnp.transpose(userFeatures)
    i_feat = jnp.transpose(itemFeatures)

    mesh = plsc.VectorSubcoreMesh(core_axis_name="c", subcore_axis_name="s")
    sc_gather = pl.kernel(
        _sc_gather_body,
        out_type=tuple(
            jax.ShapeDtypeStruct((B, EMB), jnp.float32)
            for _ in range(N_FIELDS)
        ),
        mesh=mesh,
        scratch_types=[
            pltpu.VMEM((BW,), jnp.int32),
            pltpu.VMEM((BW, EMB), jnp.float32),
            pltpu.SemaphoreType.DMA,
        ],
        compiler_params=pltpu.CompilerParams(use_tc_tiling_on_sc=False),
    )
    fields = sc_gather(user_table, item_table, u_feat, i_feat)

    idxT = jnp.transpose(indexEmb)

    grid = (B // _MLP_BLK,)
    row_spec = pl.BlockSpec((_MLP_BLK, EMB), lambda i: (i, 0))
    out = pl.pallas_call(
        _mlp_body,
        grid=grid,
        in_specs=[
            pl.BlockSpec((IDX_DIM, _MLP_BLK), lambda i: (0, i)),
            row_spec, row_spec, row_spec,
            row_spec, row_spec, row_spec, row_spec,
            pl.BlockSpec((IDX_DIM + N_FIELDS * EMB, HID), lambda i: (0, 0)),
            pl.BlockSpec((1, HID), lambda i: (0, 0)),
            pl.BlockSpec((HID, 2), lambda i: (0, 0)),
            pl.BlockSpec((1, 2), lambda i: (0, 0)),
        ],
        out_specs=pl.BlockSpec((_MLP_BLK, 2), lambda i: (i, 0)),
        out_shape=jax.ShapeDtypeStruct((B, 2), jnp.float32),
    )(idxT, *fields, W1, b1.reshape(1, HID), W2, b2.reshape(1, 2))
    return out
